# baseline (speedup 1.0000x reference)
import functools
import jax
import jax.numpy as jnp
from jax import lax
from jax.experimental import pallas as pl
from jax.experimental.pallas import tpu as pltpu
from jax.experimental.pallas import tpu_sc as plsc

B = 16384
K = 64
NC = 2
NS = 16
NW = NC * NS
NPW = B // NW
ICHUNK = 128
NCHUNK = NPW // ICHUNK


def _fm_body(user_h, item_h, uemb_h, iemb_h, ubias_h, ibias_h, gb_h, out_h,
             idx_u, idx_i, urows, irows, bu, bi, outv, gbv, tbuf, sem):
    wid = lax.axis_index("s") * NC + lax.axis_index("c")
    base = wid * NPW

    pltpu.sync_copy(user_h.at[wid], idx_u)
    pltpu.sync_copy(item_h.at[wid], idx_i)
    pltpu.sync_copy(gb_h, gbv)

    copies = []
    for j in range(NCHUNK):
        sl = pl.ds(j * ICHUNK, ICHUNK)
        copies.append(pltpu.async_copy(uemb_h.at[idx_u.at[j]], urows.at[sl], sem))
        copies.append(pltpu.async_copy(iemb_h.at[idx_i.at[j]], irows.at[sl], sem))
        copies.append(pltpu.async_copy(ubias_h.at[idx_u.at[j]], bu.at[sl], sem))
        copies.append(pltpu.async_copy(ibias_h.at[idx_i.at[j]], bi.at[sl], sem))
    for cp in copies:
        cp.wait()

    gvec = gbv[pl.ds(0, 16)]
    rows16 = jnp.arange(16, dtype=jnp.int32)

    def body(g, carry):
        base16 = g * 16
        for r in range(16):
            i = base16 + r
            acc = urows[i, pl.ds(0, 16)] * irows[i, pl.ds(0, 16)]
            for c in range(1, K // 16):
                acc = acc + urows[i, pl.ds(c * 16, 16)] * irows[i, pl.ds(c * 16, 16)]
            tbuf[r, pl.ds(0, 16)] = acc
        tot = plsc.load_gather(tbuf, [rows16, jnp.full((16,), 0, jnp.int32)])
        for k in range(1, 16):
            tot = tot + plsc.load_gather(tbuf, [rows16, jnp.full((16,), k, jnp.int32)])
        sl = pl.ds(base16, 16)
        outv[sl] = tot + bu[sl] + bi[sl] + gvec
        return carry

    lax.fori_loop(0, NPW // 16, body, 0)

    pltpu.sync_copy(outv, out_h.at[pl.ds(base, NPW)])


@jax.jit
def kernel(user, item, user_emb_table, item_emb_table, user_bias_table,
           item_bias_table, global_bias):
    user3 = user.astype(jnp.int32).reshape(NW, NCHUNK, ICHUNK)
    item3 = item.astype(jnp.int32).reshape(NW, NCHUNK, ICHUNK)
    ubias = user_bias_table.reshape(-1)
    ibias = item_bias_table.reshape(-1)
    gb16 = jnp.broadcast_to(jnp.asarray(global_bias, jnp.float32).reshape(1), (16,))

    fm = pl.kernel(
        _fm_body,
        mesh=plsc.VectorSubcoreMesh(core_axis_name="c", subcore_axis_name="s"),
        out_type=jax.ShapeDtypeStruct((B,), jnp.float32),
        compiler_params=pltpu.CompilerParams(
            needs_layout_passes=False, use_tc_tiling_on_sc=False),
        scratch_types=[
            pltpu.VMEM((NCHUNK, ICHUNK), jnp.int32),
            pltpu.VMEM((NCHUNK, ICHUNK), jnp.int32),
            pltpu.VMEM((NPW, K), jnp.float32),
            pltpu.VMEM((NPW, K), jnp.float32),
            pltpu.VMEM((NPW,), jnp.float32),
            pltpu.VMEM((NPW,), jnp.float32),
            pltpu.VMEM((NPW,), jnp.float32),
            pltpu.VMEM((16,), jnp.float32),
            pltpu.VMEM((16, 17), jnp.float32),
            pltpu.SemaphoreType.DMA,
        ],
    )
    return fm(user3, item3, user_emb_table, item_emb_table, ubias, ibias, gb16)
